# baseline (speedup 1.0000x reference)
import functools

import jax
import jax.numpy as jnp
from jax import lax
from jax.experimental import pallas as pl
from jax.experimental.pallas import tpu as pltpu
from jax.experimental.pallas import tpu_sc as plsc

VOCAB = 1_000_000
HIDDEN = 64
BATCH = 4096
HIST = 200

_B = BATCH * HIST
_NW = 32
_B_PER_W = _B // _NW
_CHUNK = 800
_NCHUNK = _B_PER_W // _CHUNK


def _make_gather():
  mesh = plsc.VectorSubcoreMesh(core_axis_name="c", subcore_axis_name="s")

  @functools.partial(
      pl.kernel,
      out_type=jax.ShapeDtypeStruct((_B, HIDDEN), jnp.float32),
      mesh=mesh,
      scratch_types=[
          pltpu.VMEM((_B_PER_W,), jnp.int32),
          pltpu.VMEM((_CHUNK, HIDDEN), jnp.float32),
          pltpu.SemaphoreType.DMA,
      ],
      compiler_params=pltpu.CompilerParams(use_tc_tiling_on_sc=False),
  )
  def gather_kernel(emb_hbm, idx_hbm, out_hbm, idx_v, rows_v, sem):
    wid = lax.axis_index("s") * 2 + lax.axis_index("c")
    base = wid * _B_PER_W
    pltpu.sync_copy(idx_hbm.at[pl.ds(base, _B_PER_W)], idx_v)

    def body(i, _):
      off = i * _CHUNK
      pltpu.async_copy(
          emb_hbm.at[idx_v.at[pl.ds(off, _CHUNK)]], rows_v, sem).wait()
      pltpu.sync_copy(rows_v, out_hbm.at[pl.ds(base + off, _CHUNK)])
      return 0

    lax.fori_loop(0, _NCHUNK, body, 0)

  return gather_kernel


_gather = _make_gather()


def kernel(input_ids, emb):
  idx = input_ids.reshape(-1).astype(jnp.int32)
  out = _gather(emb, idx)
  return out.reshape(BATCH, HIST, HIDDEN)


# speedup vs baseline: 1.0129x; 1.0129x over previous
import functools

import jax
import jax.numpy as jnp
from jax import lax
from jax.experimental import pallas as pl
from jax.experimental.pallas import tpu as pltpu
from jax.experimental.pallas import tpu_sc as plsc

VOCAB = 1_000_000
HIDDEN = 64
BATCH = 4096
HIST = 200

_B = BATCH * HIST
_NW = 32
_B_PER_W = _B // _NW
_CHUNK = 800
_NCHUNK = _B_PER_W // _CHUNK


def _make_gather():
  mesh = plsc.VectorSubcoreMesh(core_axis_name="c", subcore_axis_name="s")

  @functools.partial(
      pl.kernel,
      out_type=jax.ShapeDtypeStruct((_B, HIDDEN), jnp.float32),
      mesh=mesh,
      scratch_types=[
          pltpu.VMEM((_B_PER_W,), jnp.int32),
          pltpu.VMEM((_CHUNK, HIDDEN), jnp.float32),
          pltpu.VMEM((_CHUNK, HIDDEN), jnp.float32),
          pltpu.SemaphoreType.DMA,
          pltpu.SemaphoreType.DMA,
          pltpu.SemaphoreType.DMA,
          pltpu.SemaphoreType.DMA,
      ],
      compiler_params=pltpu.CompilerParams(use_tc_tiling_on_sc=False),
  )
  def gather_kernel(emb_hbm, idx_hbm, out_hbm, idx_v, rows0, rows1,
                    gsem0, gsem1, ssem0, ssem1):
    wid = lax.axis_index("s") * 2 + lax.axis_index("c")
    base = wid * _B_PER_W
    pltpu.sync_copy(idx_hbm.at[pl.ds(base, _B_PER_W)], idx_v)

    rows = (rows0, rows1)
    gsem = (gsem0, gsem1)
    ssem = (ssem0, ssem1)

    def gather(i, b):
      pltpu.make_async_copy(
          emb_hbm.at[idx_v.at[pl.ds(i * _CHUNK, _CHUNK)]],
          rows[b], gsem[b]).start()

    def gather_wait(b):
      pltpu.make_async_copy(
          emb_hbm.at[idx_v.at[pl.ds(0, _CHUNK)]], rows[b], gsem[b]).wait()

    def store(i, b):
      pltpu.make_async_copy(
          rows[b], out_hbm.at[pl.ds(base + i * _CHUNK, _CHUNK)],
          ssem[b]).start()

    def store_wait(b):
      pltpu.make_async_copy(
          rows[b], out_hbm.at[pl.ds(base, _CHUNK)], ssem[b]).wait()

    gather(0, 0)
    gather(1, 1)
    gather_wait(0)
    store(0, 0)

    def body(k, _):
      g = 2 + 2 * k
      for b in range(2):
        i = g + b
        store_wait(b)
        gather(i, b)
        gather_wait(1 - b)
        store(i - 1, 1 - b)
      return 0

    lax.fori_loop(0, (_NCHUNK - 2) // 2, body, 0, unroll=False)

    gather_wait(1)
    store(_NCHUNK - 1, 1)
    store_wait(0)
    store_wait(1)

  return gather_kernel


_gather = _make_gather()


def kernel(input_ids, emb):
  idx = input_ids.reshape(-1).astype(jnp.int32)
  out = _gather(emb, idx)
  return out.reshape(BATCH, HIST, HIDDEN)
